# baseline (speedup 1.0000x reference)
import functools

import jax
import jax.numpy as jnp
import numpy as np
from jax import lax
from jax.experimental import pallas as pl
from jax.experimental.pallas import tpu as pltpu
from jax.experimental.pallas import tpu_sc as plsc

_N_ROWS = 100000
_D = 256
_N_SEG = 512
_NC = 2
_NS = 16
_CH = _D // _NC
_LANE = 16
_CHUNK = 128
_OWN = (-(-_N_ROWS // _NS) + 7) // 8 * 8
_NCHUNK = -(-_OWN // _CHUNK)
_WIN = _NCHUNK * _CHUNK
_ACC_ROWS = 528
_ZROWS = _ACC_ROWS // _NS
_SEG_PER_TILE = _N_SEG // _NS

_W_OFF = np.minimum(np.arange(_NS) * _OWN, _N_ROWS - _WIN)
_ROW_IDX = _W_OFF[:, None] + np.arange(_WIN)[None, :]
_REAL_LO = np.arange(_NS) * _OWN
_REAL_HI = np.append(_REAL_LO[1:], _N_ROWS)
_OWN_MASK = (_ROW_IDX >= _REAL_LO[:, None]) & (_ROW_IDX < _REAL_HI[:, None])


def _pool_body(
    feat_hbm, ids_hbm, out_hbm, ids_v, buf_v, zrow_v, mini_v, midx_v, acc_sh,
    sem0, sem1, sem2
):
    c = lax.axis_index("c")
    s = lax.axis_index("s")
    w_off = lax.min(s * _OWN, _N_ROWS - _WIN)
    col0 = c * _CH
    zero16 = jnp.zeros((_LANE,), jnp.float32)

    def _zero(i, carry):
        zrow_v[i // 8, pl.ds((i % 8) * _LANE, _LANE)] = zero16
        return carry

    lax.fori_loop(0, _ZROWS * (_CH // _LANE), _zero, 0)
    pltpu.sync_copy(zrow_v, acc_sh.at[pl.ds(s * _ZROWS, _ZROWS)])

    def _zero_mini(i, carry):
        mini_v[i // 8, pl.ds((i % 8) * _LANE, _LANE)] = zero16
        return carry

    lax.fori_loop(0, _LANE * (_CH // _LANE), _zero_mini, 0)

    pltpu.sync_copy(ids_hbm.at[s], ids_v)

    plsc.subcore_barrier()

    def _src(j):
        row0 = pl.multiple_of(w_off + j * _CHUNK, 8)
        return feat_hbm.at[pl.ds(row0, _CHUNK), pl.ds(col0, _CH)]

    sems = (sem0, sem1, sem2)

    pltpu.async_copy(_src(0), buf_v.at[0], sem0)
    pltpu.async_copy(_src(1), buf_v.at[1], sem1)

    iota16 = lax.iota(jnp.int32, _LANE)

    def _slot(t, k):
        j = 3 * t + k
        pltpu.make_async_copy(_src(j), buf_v.at[k], sems[k]).wait()
        buf = buf_v.at[k]

        ivs = [ids_v[j, pl.ds(g * _LANE, _LANE)] for g in range(8)]
        sf = ivs[0][0]
        sl = ivs[7][_LANE - 1]
        sfv = lax.broadcast(sf, (_LANE,))
        slv = lax.broadcast(sl, (_LANE,))
        one16 = jnp.ones((_LANE,), jnp.int32)
        zero16i = jnp.zeros((_LANE,), jnp.int32)
        nbv = jnp.where(ivs[0] == sfv, one16, zero16i)
        nlv = jnp.where(ivs[0] == slv, one16, zero16i)
        for g in range(1, 8):
            nbv = nbv + jnp.where(ivs[g] == sfv, one16, zero16i)
            nlv = nlv + jnp.where(ivs[g] == slv, one16, zero16i)
        nb = jnp.sum(nbv)
        fast = jnp.logical_or((nb + jnp.sum(nlv)) == _CHUNK, sf == sl)

        @pl.when(fast)
        def _():
            zeros8 = tuple(jnp.zeros((_LANE,), jnp.float32) for _ in range(8))

            def _rsum(i, accs):
                return tuple(
                    a + buf[i, pl.ds(kk * _LANE, _LANE)]
                    for kk, a in enumerate(accs)
                )

            def _ranged_sum(lo, hi, accs):
                n4 = (hi - lo) // 4

                def _b4(i, a):
                    r = lo + 4 * i
                    for d in range(4):
                        a = _rsum(r + d, a)
                    return a

                accs = lax.fori_loop(0, n4, _b4, accs)
                return lax.fori_loop(lo + 4 * n4, hi, _rsum, accs)

            run_a = _ranged_sum(0, nb, zeros8)
            run_b = _ranged_sum(nb, _CHUNK, zeros8)
            for kk in range(8):
                mini_v[0, pl.ds(kk * _LANE, _LANE)] = run_a[kk]
                mini_v[1, pl.ds(kk * _LANE, _LANE)] = run_b[kk]
            idx = jnp.where(
                iota16 == 0, sfv, jnp.where(iota16 == 1, slv, _N_SEG)
            )
            midx_v[0, pl.ds(0, _LANE)] = idx
            pltpu.sync_copy(mini_v, acc_sh.at[midx_v.at[0]], add=True)

        @pl.when(jnp.logical_not(fast))
        def _():
            pltpu.sync_copy(buf, acc_sh.at[ids_v.at[j]], add=True)

        @pl.when(j + 2 < _NCHUNK)
        def _():
            kn = (k + 2) % 3
            pltpu.async_copy(_src(j + 2), buf_v.at[kn], sems[kn])

    def _trio(t, carry):
        _slot(t, 0)
        _slot(t, 1)
        _slot(t, 2)
        return carry

    lax.fori_loop(0, (_NCHUNK - 1) // 3, _trio, 0)
    _slot((_NCHUNK - 1) // 3, 0)

    plsc.subcore_barrier()
    pltpu.sync_copy(
        acc_sh.at[pl.ds(s * _SEG_PER_TILE, _SEG_PER_TILE)],
        out_hbm.at[
            pl.ds(s * _SEG_PER_TILE, _SEG_PER_TILE), pl.ds(col0, _CH)
        ],
    )


_pool = pl.kernel(
    _pool_body,
    out_type=jax.ShapeDtypeStruct((_N_SEG, _D), jnp.float32),
    mesh=plsc.VectorSubcoreMesh(core_axis_name="c", subcore_axis_name="s"),
    scratch_types=[
        pltpu.VMEM((_NCHUNK, _CHUNK), jnp.int32),
        pltpu.VMEM((3, _CHUNK, _CH), jnp.float32),
        pltpu.VMEM((_ZROWS, _CH), jnp.float32),
        pltpu.VMEM((_LANE, _CH), jnp.float32),
        pltpu.VMEM((1, _LANE), jnp.int32),
        pltpu.VMEM_SHARED((_ACC_ROWS, _CH), jnp.float32),
        pltpu.SemaphoreType.DMA,
        pltpu.SemaphoreType.DMA,
        pltpu.SemaphoreType.DMA,
    ],
    compiler_params=pltpu.CompilerParams(needs_layout_passes=False),
)


@jax.jit
def kernel(feat, segment_ids):
    ids = segment_ids.astype(jnp.int32)
    win = jnp.stack(
        [lax.slice(ids, (int(o),), (int(o) + _WIN,)) for o in _W_OFF]
    )
    ids_padded = jnp.where(jnp.asarray(_OWN_MASK), win, _N_SEG).reshape(
        _NS, _NCHUNK, _CHUNK
    )
    return _pool(feat, ids_padded)


# speedup vs baseline: 1.0301x; 1.0301x over previous
import functools

import jax
import jax.numpy as jnp
import numpy as np
from jax import lax
from jax.experimental import pallas as pl
from jax.experimental.pallas import tpu as pltpu
from jax.experimental.pallas import tpu_sc as plsc

_N_ROWS = 100000
_D = 256
_N_SEG = 512
_NC = 2
_NS = 16
_CH = _D // _NC
_LANE = 16
_CHUNK = 128
_OWN = (-(-_N_ROWS // _NS) + 7) // 8 * 8
_NCHUNK = -(-_OWN // _CHUNK)
_WIN = _NCHUNK * _CHUNK
_ACC_ROWS = 528
_ZROWS = _ACC_ROWS // _NS
_SEG_PER_TILE = _N_SEG // _NS

_W_OFF = np.minimum(np.arange(_NS) * _OWN, _N_ROWS - _WIN)
_ROW_IDX = _W_OFF[:, None] + np.arange(_WIN)[None, :]
_REAL_LO = np.arange(_NS) * _OWN
_REAL_HI = np.append(_REAL_LO[1:], _N_ROWS)
_OWN_MASK = (_ROW_IDX >= _REAL_LO[:, None]) & (_ROW_IDX < _REAL_HI[:, None])


def _pool_body(
    feat_hbm, ids_hbm, out_hbm, ids_v, buf_v, zrow_v, mini_v, midx_v, acc_sh,
    sem0, sem1, sem2
):
    c = lax.axis_index("c")
    s = lax.axis_index("s")
    w_off = lax.min(s * _OWN, _N_ROWS - _WIN)
    col0 = c * _CH
    zero16 = jnp.zeros((_LANE,), jnp.float32)

    def _zero(i, carry):
        zrow_v[i // 8, pl.ds((i % 8) * _LANE, _LANE)] = zero16
        return carry

    lax.fori_loop(0, _ZROWS * (_CH // _LANE), _zero, 0)
    pltpu.sync_copy(zrow_v, acc_sh.at[pl.ds(s * _ZROWS, _ZROWS)])

    def _zero_mini(i, carry):
        mini_v[i // 8, pl.ds((i % 8) * _LANE, _LANE)] = zero16
        return carry

    lax.fori_loop(0, _LANE * (_CH // _LANE), _zero_mini, 0)

    pltpu.sync_copy(ids_hbm.at[s], ids_v)

    plsc.subcore_barrier()

    def _src(j):
        row0 = pl.multiple_of(w_off + j * _CHUNK, 8)
        return feat_hbm.at[pl.ds(row0, _CHUNK), pl.ds(col0, _CH)]

    sems = (sem0, sem1, sem2)

    pltpu.async_copy(_src(0), buf_v.at[0], sem0)
    pltpu.async_copy(_src(1), buf_v.at[1], sem1)

    iota16 = lax.iota(jnp.int32, _LANE)
    dummy16 = jnp.full((_LANE,), _N_SEG, jnp.int32)

    def _slot(t, k, idxv):
        j = 3 * t + k
        pltpu.make_async_copy(_src(j), buf_v.at[k], sems[k]).wait()
        buf = buf_v.at[k]

        ivs = [ids_v[j, pl.ds(g * _LANE, _LANE)] for g in range(8)]
        sf = ivs[0][0]
        sl = ivs[7][_LANE - 1]
        sfv = lax.broadcast(sf, (_LANE,))
        slv = lax.broadcast(sl, (_LANE,))
        one16 = jnp.ones((_LANE,), jnp.int32)
        zero16i = jnp.zeros((_LANE,), jnp.int32)
        nbv = jnp.where(ivs[0] == sfv, one16, zero16i)
        nlv = jnp.where(ivs[0] == slv, one16, zero16i)
        for g in range(1, 8):
            nbv = nbv + jnp.where(ivs[g] == sfv, one16, zero16i)
            nlv = nlv + jnp.where(ivs[g] == slv, one16, zero16i)
        nb = jnp.sum(nbv)
        fast = jnp.logical_or((nb + jnp.sum(nlv)) == _CHUNK, sf == sl)

        @pl.when(fast)
        def _():
            zeros8 = tuple(jnp.zeros((_LANE,), jnp.float32) for _ in range(8))

            def _rsum(i, accs):
                return tuple(
                    a + buf[i, pl.ds(kk * _LANE, _LANE)]
                    for kk, a in enumerate(accs)
                )

            def _ranged_sum(lo, hi, accs):
                n4 = (hi - lo) // 4

                def _b4(i, a):
                    r = lo + 4 * i
                    for d in range(4):
                        a = _rsum(r + d, a)
                    return a

                accs = lax.fori_loop(0, n4, _b4, accs)
                return lax.fori_loop(lo + 4 * n4, hi, _rsum, accs)

            run_a = _ranged_sum(0, nb, zeros8)
            run_b = _ranged_sum(nb, _CHUNK, zeros8)
            for kk in range(8):
                mini_v[2 * k, pl.ds(kk * _LANE, _LANE)] = run_a[kk]
                mini_v[2 * k + 1, pl.ds(kk * _LANE, _LANE)] = run_b[kk]

        @pl.when(jnp.logical_not(fast))
        def _():
            pltpu.sync_copy(buf, acc_sh.at[ids_v.at[j]], add=True)

        @pl.when(j + 2 < _NCHUNK)
        def _():
            kn = (k + 2) % 3
            pltpu.async_copy(_src(j + 2), buf_v.at[kn], sems[kn])

        a = jnp.where(fast, sfv, dummy16)
        b = jnp.where(fast, slv, dummy16)
        return jnp.where(
            iota16 == 2 * k, a, jnp.where(iota16 == 2 * k + 1, b, idxv)
        )

    def _flush(idxv):
        midx_v[0, pl.ds(0, _LANE)] = idxv
        pltpu.sync_copy(mini_v, acc_sh.at[midx_v.at[0]], add=True)

    def _trio(t, carry):
        idxv = dummy16
        idxv = _slot(t, 0, idxv)
        idxv = _slot(t, 1, idxv)
        idxv = _slot(t, 2, idxv)
        _flush(idxv)
        return carry

    lax.fori_loop(0, (_NCHUNK - 1) // 3, _trio, 0)
    _flush(_slot((_NCHUNK - 1) // 3, 0, dummy16))

    plsc.subcore_barrier()
    pltpu.sync_copy(
        acc_sh.at[pl.ds(s * _SEG_PER_TILE, _SEG_PER_TILE)],
        out_hbm.at[
            pl.ds(s * _SEG_PER_TILE, _SEG_PER_TILE), pl.ds(col0, _CH)
        ],
    )


_pool = pl.kernel(
    _pool_body,
    out_type=jax.ShapeDtypeStruct((_N_SEG, _D), jnp.float32),
    mesh=plsc.VectorSubcoreMesh(core_axis_name="c", subcore_axis_name="s"),
    scratch_types=[
        pltpu.VMEM((_NCHUNK, _CHUNK), jnp.int32),
        pltpu.VMEM((3, _CHUNK, _CH), jnp.float32),
        pltpu.VMEM((_ZROWS, _CH), jnp.float32),
        pltpu.VMEM((_LANE, _CH), jnp.float32),
        pltpu.VMEM((1, _LANE), jnp.int32),
        pltpu.VMEM_SHARED((_ACC_ROWS, _CH), jnp.float32),
        pltpu.SemaphoreType.DMA,
        pltpu.SemaphoreType.DMA,
        pltpu.SemaphoreType.DMA,
    ],
    compiler_params=pltpu.CompilerParams(needs_layout_passes=False),
)


@jax.jit
def kernel(feat, segment_ids):
    ids = segment_ids.astype(jnp.int32)
    win = jnp.stack(
        [lax.slice(ids, (int(o),), (int(o) + _WIN,)) for o in _W_OFF]
    )
    ids_padded = jnp.where(jnp.asarray(_OWN_MASK), win, _N_SEG).reshape(
        _NS, _NCHUNK, _CHUNK
    )
    return _pool(feat, ids_padded)


# speedup vs baseline: 1.1860x; 1.1514x over previous
import functools

import jax
import jax.numpy as jnp
import numpy as np
from jax import lax
from jax.experimental import pallas as pl
from jax.experimental.pallas import tpu as pltpu
from jax.experimental.pallas import tpu_sc as plsc

_N_ROWS = 100000
_D = 256
_N_SEG = 512
_NC = 2
_NS = 16
_CH = _D // _NC
_LANE = 16
_CHUNK = 128
_OWN = (-(-_N_ROWS // _NS) + 7) // 8 * 8
_NCHUNK = -(-_OWN // _CHUNK)
_WIN = _NCHUNK * _CHUNK
_ACC_ROWS = 528
_ZROWS = _ACC_ROWS // _NS
_SEG_PER_TILE = _N_SEG // _NS

_W_OFF = np.minimum(np.arange(_NS) * _OWN, _N_ROWS - _WIN)
_ROW_IDX = _W_OFF[:, None] + np.arange(_WIN)[None, :]
_REAL_LO = np.arange(_NS) * _OWN
_REAL_HI = np.append(_REAL_LO[1:], _N_ROWS)
_OWN_MASK = (_ROW_IDX >= _REAL_LO[:, None]) & (_ROW_IDX < _REAL_HI[:, None])


def _pool_body(
    feat_hbm, ids_hbm, out_hbm, ids_v, buf_v, zrow_v, mini_v, midx_v, acc_sh,
    sem0, sem1, sem2, sem3
):
    c = lax.axis_index("c")
    s = lax.axis_index("s")
    w_off = lax.min(s * _OWN, _N_ROWS - _WIN)
    col0 = c * _CH
    zero16 = jnp.zeros((_LANE,), jnp.float32)

    def _zero(i, carry):
        zrow_v[i // 8, pl.ds((i % 8) * _LANE, _LANE)] = zero16
        return carry

    lax.fori_loop(0, _ZROWS * (_CH // _LANE), _zero, 0)
    pltpu.sync_copy(zrow_v, acc_sh.at[pl.ds(s * _ZROWS, _ZROWS)])

    def _zero_mini(i, carry):
        mini_v[i // 8, pl.ds((i % 8) * _LANE, _LANE)] = zero16
        return carry

    lax.fori_loop(0, _LANE * (_CH // _LANE), _zero_mini, 0)

    pltpu.sync_copy(ids_hbm.at[s], ids_v)

    plsc.subcore_barrier()

    def _src(j):
        row0 = pl.multiple_of(w_off + j * _CHUNK, 8)
        return feat_hbm.at[pl.ds(row0, _CHUNK), pl.ds(col0, _CH)]

    sems = (sem0, sem1, sem2, sem3)

    pltpu.async_copy(_src(0), buf_v.at[0], sem0)
    pltpu.async_copy(_src(1), buf_v.at[1], sem1)
    pltpu.async_copy(_src(2), buf_v.at[2], sem2)

    iota16 = lax.iota(jnp.int32, _LANE)
    dummy16 = jnp.full((_LANE,), _N_SEG, jnp.int32)

    def _slot(t, k, idxv):
        j = 4 * t + k
        pltpu.make_async_copy(_src(j), buf_v.at[k], sems[k]).wait()
        buf = buf_v.at[k]

        ivs = [ids_v[j, pl.ds(g * _LANE, _LANE)] for g in range(8)]
        sf = ivs[0][0]
        sl = ivs[7][_LANE - 1]
        sfv = lax.broadcast(sf, (_LANE,))
        slv = lax.broadcast(sl, (_LANE,))
        one16 = jnp.ones((_LANE,), jnp.int32)
        zero16i = jnp.zeros((_LANE,), jnp.int32)
        nbv = jnp.where(ivs[0] == sfv, one16, zero16i)
        nlv = jnp.where(ivs[0] == slv, one16, zero16i)
        for g in range(1, 8):
            nbv = nbv + jnp.where(ivs[g] == sfv, one16, zero16i)
            nlv = nlv + jnp.where(ivs[g] == slv, one16, zero16i)
        nb = jnp.sum(nbv)
        fast = jnp.logical_or((nb + jnp.sum(nlv)) == _CHUNK, sf == sl)

        @pl.when(fast)
        def _():
            zeros8 = tuple(jnp.zeros((_LANE,), jnp.float32) for _ in range(8))

            def _rsum(i, accs):
                return tuple(
                    a + buf[i, pl.ds(kk * _LANE, _LANE)]
                    for kk, a in enumerate(accs)
                )

            def _ranged_sum(lo, hi, accs):
                n4 = (hi - lo) // 4

                def _b4(i, a):
                    r = lo + 4 * i
                    for d in range(4):
                        a = _rsum(r + d, a)
                    return a

                accs = lax.fori_loop(0, n4, _b4, accs)
                return lax.fori_loop(lo + 4 * n4, hi, _rsum, accs)

            run_a = _ranged_sum(0, nb, zeros8)
            run_b = _ranged_sum(nb, _CHUNK, zeros8)
            for kk in range(8):
                mini_v[2 * k, pl.ds(kk * _LANE, _LANE)] = run_a[kk]
                mini_v[2 * k + 1, pl.ds(kk * _LANE, _LANE)] = run_b[kk]

        @pl.when(jnp.logical_not(fast))
        def _():
            pltpu.sync_copy(buf, acc_sh.at[ids_v.at[j]], add=True)

        @pl.when(j + 3 < _NCHUNK)
        def _():
            kn = (k + 3) % 4
            pltpu.async_copy(_src(j + 3), buf_v.at[kn], sems[kn])

        a = jnp.where(fast, sfv, dummy16)
        b = jnp.where(fast, slv, dummy16)
        return jnp.where(
            iota16 == 2 * k, a, jnp.where(iota16 == 2 * k + 1, b, idxv)
        )

    def _flush(idxv):
        midx_v[0, pl.ds(0, _LANE)] = idxv
        pltpu.sync_copy(mini_v, acc_sh.at[midx_v.at[0]], add=True)

    def _quad(t, carry):
        idxv = dummy16
        idxv = _slot(t, 0, idxv)
        idxv = _slot(t, 1, idxv)
        idxv = _slot(t, 2, idxv)
        idxv = _slot(t, 3, idxv)
        _flush(idxv)
        return carry

    lax.fori_loop(0, (_NCHUNK - 1) // 4, _quad, 0)
    _flush(_slot((_NCHUNK - 1) // 4, 0, dummy16))

    plsc.subcore_barrier()
    pltpu.sync_copy(
        acc_sh.at[pl.ds(s * _SEG_PER_TILE, _SEG_PER_TILE)],
        out_hbm.at[
            pl.ds(s * _SEG_PER_TILE, _SEG_PER_TILE), pl.ds(col0, _CH)
        ],
    )


_pool = pl.kernel(
    _pool_body,
    out_type=jax.ShapeDtypeStruct((_N_SEG, _D), jnp.float32),
    mesh=plsc.VectorSubcoreMesh(core_axis_name="c", subcore_axis_name="s"),
    scratch_types=[
        pltpu.VMEM((_NCHUNK, _CHUNK), jnp.int32),
        pltpu.VMEM((4, _CHUNK, _CH), jnp.float32),
        pltpu.VMEM((_ZROWS, _CH), jnp.float32),
        pltpu.VMEM((_LANE, _CH), jnp.float32),
        pltpu.VMEM((1, _LANE), jnp.int32),
        pltpu.VMEM_SHARED((_ACC_ROWS, _CH), jnp.float32),
        pltpu.SemaphoreType.DMA,
        pltpu.SemaphoreType.DMA,
        pltpu.SemaphoreType.DMA,
        pltpu.SemaphoreType.DMA,
    ],
    compiler_params=pltpu.CompilerParams(needs_layout_passes=False),
)


@jax.jit
def kernel(feat, segment_ids):
    ids = segment_ids.astype(jnp.int32)
    win = jnp.stack(
        [lax.slice(ids, (int(o),), (int(o) + _WIN,)) for o in _W_OFF]
    )
    ids_padded = jnp.where(jnp.asarray(_OWN_MASK), win, _N_SEG).reshape(
        _NS, _NCHUNK, _CHUNK
    )
    return _pool(feat, ids_padded)
